# Initial kernel scaffold; baseline (speedup 1.0000x reference)
import jax
import jax.numpy as jnp
from jax.experimental import pallas as pl


def kernel(adj_indices, adj_values, A_pv, A_vp, A_pc, A_cp, A_cv, A_vc, embedding, pri_emb, cate_emb, mat_pv, mat_pc, mat_cp, mat_cv, Wi, bi, Wi1, bi1, Wi2, bi2, Wp, bp, Wp1, bp1, Wp2, bp2, Wc, bc, Wc1, bc1, Wc2, bc2):
    raise NotImplementedError("write your pallas kernel here")


# speedup vs baseline: 6.5809x; 6.5809x over previous
import functools

import jax
import jax.numpy as jnp
from jax import lax
from jax.experimental import pallas as pl
from jax.experimental.pallas import tpu as pltpu
from jax.experimental.pallas import tpu_sc as plsc

EMB = 64
N_NODE = 10000
N_PRICE = 100
N_CAT = 500
N_EDGES = 640000

_NC = 2
_NS = 16
_NW = _NC * _NS
_EB = 128
_NBLK = -(-N_EDGES // _EB)
_BLK_W = (-(-_NBLK // _NW) + 7) // 8 * 8
_NBLK_PAD = _BLK_W * _NW
_NE_PAD = _NBLK_PAD * _EB
_NPN = 10240
_RPT = _NPN // _NS
_ZR = 128
_BLK_H = _BLK_W // 2


def _spmm_body(src_hbm, dst_hbm, val_hbm, emb_hbm, out_hbm,
               acc_spm, src_v, dst_v, val_v, rows_v):
    cid = lax.axis_index("c")
    sid = lax.axis_index("s")
    wid = cid * _NS + sid
    base = sid * _RPT

    def _zero_row(r, carry):
        for q in range(EMB // 16):
            rows_v[r, pl.ds(q * 16, 16)] = jnp.zeros((16,), jnp.float32)
        return carry

    lax.fori_loop(0, _ZR, _zero_row, 0)
    for r in range(_RPT // _ZR):
        pltpu.sync_copy(rows_v, acc_spm.at[pl.ds(base + r * _ZR, _ZR)])

    row0 = wid * _BLK_W

    plsc.subcore_barrier()

    def _row(b, carry):
        pltpu.sync_copy(emb_hbm.at[src_v.at[b]], rows_v)
        pltpu.sync_copy(val_hbm.at[carry + b], val_v)
        for i in range(_EB):
            sv = val_v[i]
            for q in range(EMB // 16):
                rows_v[i, pl.ds(q * 16, 16)] = (
                    rows_v[i, pl.ds(q * 16, 16)] * sv)
        pltpu.sync_copy(rows_v, acc_spm.at[dst_v.at[b]], add=True)
        return carry

    for h in range(_BLK_W // _BLK_H):
        h0 = row0 + h * _BLK_H
        pltpu.sync_copy(src_hbm.at[pl.ds(h0, _BLK_H)], src_v)
        pltpu.sync_copy(dst_hbm.at[pl.ds(h0, _BLK_H)], dst_v)
        lax.fori_loop(0, _BLK_H, _row, h0)

    plsc.subcore_barrier()
    pltpu.sync_copy(acc_spm.at[pl.ds(base, _RPT)],
                    out_hbm.at[cid, pl.ds(base, _RPT)])


@functools.cache
def _get_spmm_sc():
    return pl.kernel(
        _spmm_body,
        out_type=jax.ShapeDtypeStruct((_NC, _NPN, EMB), jnp.float32),
        mesh=plsc.VectorSubcoreMesh(core_axis_name="c", subcore_axis_name="s",
                                    num_cores=_NC, num_subcores=_NS),
        compiler_params=pltpu.CompilerParams(use_tc_tiling_on_sc=False),
        scratch_types=[
            pltpu.VMEM_SHARED((_NPN, EMB), jnp.float32),
            pltpu.VMEM((_BLK_H, _EB), jnp.int32),
            pltpu.VMEM((_BLK_H, _EB), jnp.int32),
            pltpu.VMEM((_EB, 16), jnp.float32),
            pltpu.VMEM((_EB, EMB), jnp.float32),
        ],
    )


def _spmm_sc(src, dst, val, e):
    return _get_spmm_sc()(src, dst, val, e)


_BI = 2048
_GI = -(-N_NODE // _BI)
_BJ = 2048
_GJ = -(-N_NODE // _BJ)


def _t1_body(e, avp, avc, p, c, Wi, bi, Wi1, bi1, Wi2, bi2, pt0, pt1, out):
    eb = e[...]
    evp = jnp.dot(avp[...], p[...], preferred_element_type=jnp.float32)
    evc = jnp.dot(avc[...], c[...], preferred_element_type=jnp.float32)
    W = Wi[...]
    z = (jnp.dot(eb, W[0:EMB], preferred_element_type=jnp.float32)
         + jnp.dot(evp, W[EMB:2 * EMB] + Wi1[...],
                   preferred_element_type=jnp.float32)
         + jnp.dot(evc, W[2 * EMB:3 * EMB] + Wi2[...],
                   preferred_element_type=jnp.float32)
         + bi[...] + bi1[...] + bi2[...])
    g = jax.nn.sigmoid(z)
    out[...] = eb + g * evp + (1.0 - g) * evc + pt0[0] + pt1[0]


def _t1(e, p, c, avp, avc, Wi, bi, Wi1, bi1, Wi2, bi2, parts):
    full2 = lambda shape: pl.BlockSpec(shape, lambda i: (0, 0))
    return pl.pallas_call(
        _t1_body,
        grid=(_GI,),
        in_specs=[
            pl.BlockSpec((_BI, EMB), lambda i: (i, 0)),
            pl.BlockSpec((_BI, N_PRICE), lambda i: (i, 0)),
            pl.BlockSpec((_BI, N_CAT), lambda i: (i, 0)),
            full2((N_PRICE, EMB)),
            full2((N_CAT, EMB)),
            full2((3 * EMB, EMB)),
            full2((1, EMB)),
            full2((EMB, EMB)),
            full2((1, EMB)),
            full2((EMB, EMB)),
            full2((1, EMB)),
            pl.BlockSpec((1, _BI, EMB), lambda i: (0, i, 0)),
            pl.BlockSpec((1, _BI, EMB), lambda i: (1, i, 0)),
        ],
        out_specs=pl.BlockSpec((_BI, EMB), lambda i: (i, 0)),
        out_shape=jax.ShapeDtypeStruct((N_NODE, EMB), jnp.float32),
    )(e, avp, avc, p, c, Wi, bi.reshape(1, EMB), Wi1, bi1.reshape(1, EMB),
      Wi2, bi2.reshape(1, EMB), parts, parts)


def _t0_body(e, s_out, smax_out, smin_out, mx, mn):
    j = pl.program_id(0)
    row_ids = jax.lax.broadcasted_iota(jnp.int32, (_BJ, 1), 0) + j * _BJ
    eb = jnp.where(row_ids < N_NODE, e[...], 0.0)
    ones = jnp.ones((1, EMB), jnp.float32)
    s1 = lax.dot_general(ones, eb, (((1,), (1,)), ((), ())),
                         preferred_element_type=jnp.float32)
    s_out[...] = s1
    col_ok = (jax.lax.broadcasted_iota(jnp.int32, (1, _BJ), 1) + j * _BJ) < N_NODE
    bmax = jnp.max(jnp.where(col_ok, s1, -jnp.inf))
    bmin = jnp.min(jnp.where(col_ok, s1, jnp.inf))

    @pl.when(j == 0)
    def _():
        mx[0] = bmax
        mn[0] = bmin

    @pl.when(j > 0)
    def _():
        mx[0] = jnp.maximum(mx[0], bmax)
        mn[0] = jnp.minimum(mn[0], bmin)

    @pl.when(j == _GJ - 1)
    def _():
        smax_out[0, 0] = mx[0]
        smin_out[0, 0] = mn[0]


def _t0(e):
    return pl.pallas_call(
        _t0_body,
        grid=(_GJ,),
        in_specs=[pl.BlockSpec((_BJ, EMB), lambda j: (j, 0))],
        out_specs=[
            pl.BlockSpec((1, _BJ), lambda j: (0, j)),
            pl.BlockSpec(memory_space=pltpu.SMEM),
            pl.BlockSpec(memory_space=pltpu.SMEM),
        ],
        out_shape=[
            jax.ShapeDtypeStruct((1, N_NODE), jnp.float32),
            jax.ShapeDtypeStruct((1, 1), jnp.float32),
            jax.ShapeDtypeStruct((1, 1), jnp.float32),
        ],
        scratch_shapes=[
            pltpu.SMEM((1,), jnp.float32),
            pltpu.SMEM((1,), jnp.float32),
        ],
    )(e)


def _intra_small(A, mat, s_row, e2):
    m = jnp.where(mat > 0, mat * jnp.max(s_row), mat * jnp.min(s_row))
    w = jnp.exp(mat * s_row - m)
    b = w * A
    num = jnp.dot(b, e2, preferred_element_type=jnp.float32)
    den = (jnp.sum(b, axis=1, keepdims=True)
           + 1e-8 * jnp.sum(w, axis=1, keepdims=True))
    return num / den


def _gate_combine(e1, e2, e3, W, b0, W1, b1, W2, b2):
    z = (jnp.dot(e1, W[0:EMB], preferred_element_type=jnp.float32)
         + jnp.dot(e2, W[EMB:2 * EMB] + W1, preferred_element_type=jnp.float32)
         + jnp.dot(e3, W[2 * EMB:3 * EMB] + W2,
                   preferred_element_type=jnp.float32)
         + b0 + b1 + b2)
    g = jax.nn.sigmoid(z)
    return e1 + g * e2 + (1.0 - g) * e3


def _t2_body(apv, acv, e, s, smax, smin, apc, acp, p, c, mpv, mpc, mcp, mcv,
             Wp, bp, Wp1, bp1, Wp2, bp2, Wc, bc, Wc1, bc1, Wc2, bc2,
             price, cate, npv, dpv, spv, ncv, dcv, scv):
    j = pl.program_id(0)

    @pl.when(j == 0)
    def _():
        npv[...] = jnp.zeros_like(npv)
        dpv[...] = jnp.zeros_like(dpv)
        spv[...] = jnp.zeros_like(spv)
        ncv[...] = jnp.zeros_like(ncv)
        dcv[...] = jnp.zeros_like(dcv)
        scv[...] = jnp.zeros_like(scv)

    row_ids = jax.lax.broadcasted_iota(jnp.int32, (_BJ, 1), 0) + j * _BJ
    row_ok = row_ids < N_NODE
    eb = jnp.where(row_ok, e[...], 0.0)
    col_ok = (jax.lax.broadcasted_iota(jnp.int32, (1, _BJ), 1) + j * _BJ) < N_NODE
    s1 = s[...]
    sx = smax[0, 0]
    sn = smin[0, 0]

    m_pv = jnp.where(mpv[...] > 0, mpv[...] * sx, mpv[...] * sn)
    w_pv = jnp.where(col_ok, jnp.exp(mpv[...] * s1 - m_pv), 0.0)
    b_pv = jnp.where(col_ok, w_pv * apv[...], 0.0)
    npv[...] += jnp.dot(b_pv, eb, preferred_element_type=jnp.float32)
    dpv[...] += jnp.sum(b_pv, axis=1, keepdims=True)
    spv[...] += jnp.sum(w_pv, axis=1, keepdims=True)

    m_cv = jnp.where(mcv[...] > 0, mcv[...] * sx, mcv[...] * sn)
    w_cv = jnp.where(col_ok, jnp.exp(mcv[...] * s1 - m_cv), 0.0)
    b_cv = jnp.where(col_ok, w_cv * acv[...], 0.0)
    ncv[...] += jnp.dot(b_cv, eb, preferred_element_type=jnp.float32)
    dcv[...] += jnp.sum(b_cv, axis=1, keepdims=True)
    scv[...] += jnp.sum(w_cv, axis=1, keepdims=True)

    @pl.when(j == _GJ - 1)
    def _():
        ip_v = npv[...] / (dpv[...] + 1e-8 * spv[...])
        ic_v = ncv[...] / (dcv[...] + 1e-8 * scv[...])
        pb = p[...]
        cb = c[...]
        onesE = jnp.ones((1, EMB), jnp.float32)
        s_c = lax.dot_general(onesE, cb, (((1,), (1,)), ((), ())),
                              preferred_element_type=jnp.float32)
        s_p = lax.dot_general(onesE, pb, (((1,), (1,)), ((), ())),
                              preferred_element_type=jnp.float32)
        ip_c = _intra_small(apc[...], mpc[...], s_c, cb)
        ic_p = _intra_small(acp[...], mcp[...], s_p, pb)
        price[...] = _gate_combine(pb, ip_v, ip_c, Wp[...], bp[...],
                                   Wp1[...], bp1[...], Wp2[...], bp2[...])
        cate[...] = _gate_combine(cb, ic_p, ic_v, Wc[...], bc[...],
                                  Wc1[...], bc1[...], Wc2[...], bc2[...])


def _t2(e, s, smax, smin, p, c, apv, apc, acp, acv, mpv, mpc, mcp, mcv,
        Wp, bp, Wp1, bp1, Wp2, bp2, Wc, bc, Wc1, bc1, Wc2, bc2):
    full2 = lambda shape: pl.BlockSpec(shape, lambda j: (0, 0))
    return pl.pallas_call(
        _t2_body,
        grid=(_GJ,),
        in_specs=[
            pl.BlockSpec((N_PRICE, _BJ), lambda j: (0, j)),
            pl.BlockSpec((N_CAT, _BJ), lambda j: (0, j)),
            pl.BlockSpec((_BJ, EMB), lambda j: (j, 0)),
            pl.BlockSpec((1, _BJ), lambda j: (0, j)),
            pl.BlockSpec(memory_space=pltpu.SMEM),
            pl.BlockSpec(memory_space=pltpu.SMEM),
            full2((N_PRICE, N_CAT)),
            full2((N_CAT, N_PRICE)),
            full2((N_PRICE, EMB)),
            full2((N_CAT, EMB)),
            full2((N_PRICE, 1)),
            full2((N_PRICE, 1)),
            full2((N_CAT, 1)),
            full2((N_CAT, 1)),
            full2((3 * EMB, EMB)),
            full2((1, EMB)),
            full2((EMB, EMB)),
            full2((1, EMB)),
            full2((EMB, EMB)),
            full2((1, EMB)),
            full2((3 * EMB, EMB)),
            full2((1, EMB)),
            full2((EMB, EMB)),
            full2((1, EMB)),
            full2((EMB, EMB)),
            full2((1, EMB)),
        ],
        out_specs=[
            pl.BlockSpec((N_PRICE, EMB), lambda j: (0, 0)),
            pl.BlockSpec((N_CAT, EMB), lambda j: (0, 0)),
        ],
        out_shape=[
            jax.ShapeDtypeStruct((N_PRICE, EMB), jnp.float32),
            jax.ShapeDtypeStruct((N_CAT, EMB), jnp.float32),
        ],
        scratch_shapes=[
            pltpu.VMEM((N_PRICE, EMB), jnp.float32),
            pltpu.VMEM((N_PRICE, 1), jnp.float32),
            pltpu.VMEM((N_PRICE, 1), jnp.float32),
            pltpu.VMEM((N_CAT, EMB), jnp.float32),
            pltpu.VMEM((N_CAT, 1), jnp.float32),
            pltpu.VMEM((N_CAT, 1), jnp.float32),
        ],
    )(apv, acv, e, s, smax, smin, apc, acp, p, c, mpv, mpc, mcp, mcv,
      Wp, bp.reshape(1, EMB), Wp1, bp1.reshape(1, EMB), Wp2,
      bp2.reshape(1, EMB), Wc, bc.reshape(1, EMB), Wc1, bc1.reshape(1, EMB),
      Wc2, bc2.reshape(1, EMB))


def kernel(adj_indices, adj_values, A_pv, A_vp, A_pc, A_cp, A_cv, A_vc,
           embedding, pri_emb, cate_emb, mat_pv, mat_pc, mat_cp, mat_cv,
           Wi, bi, Wi1, bi1, Wi2, bi2, Wp, bp, Wp1, bp1, Wp2, bp2,
           Wc, bc, Wc1, bc1, Wc2, bc2):
    npad = _NE_PAD - N_EDGES
    pad_idx = jnp.arange(npad, dtype=jnp.int32) % N_NODE
    src = jnp.concatenate([adj_indices[1], pad_idx]).reshape(_NBLK_PAD, _EB)
    dst = jnp.concatenate([adj_indices[0], pad_idx]).reshape(_NBLK_PAD, _EB)
    val = jnp.broadcast_to(
        jnp.concatenate([adj_values, jnp.zeros((npad,), jnp.float32)])
        .reshape(_NBLK_PAD, _EB)[:, :, None], (_NBLK_PAD, _EB, 16))

    e, p, c = embedding, pri_emb, cate_emb
    item = price = None
    for _ in range(2):
        e_pad = jnp.concatenate(
            [e, jnp.zeros((_NPN - N_NODE, EMB), jnp.float32)])
        parts = _spmm_sc(src, dst, val, e_pad)
        s, smax, smin = _t0(e)
        price, cate = _t2(e, s, smax, smin, p, c, A_pv, A_pc, A_cp, A_cv,
                          mat_pv, mat_pc, mat_cp, mat_cv,
                          Wp, bp, Wp1, bp1, Wp2, bp2,
                          Wc, bc, Wc1, bc1, Wc2, bc2)
        item = _t1(e, p, c, A_vp, A_vc, Wi, bi, Wi1, bi1, Wi2, bi2, parts)
        e, p, c = item, price, cate
    return (item, price)


# speedup vs baseline: 9.1358x; 1.3882x over previous
import functools

import jax
import jax.numpy as jnp
from jax import lax
from jax.experimental import pallas as pl
from jax.experimental.pallas import tpu as pltpu
from jax.experimental.pallas import tpu_sc as plsc

EMB = 64
N_NODE = 10000
N_PRICE = 100
N_CAT = 500
N_EDGES = 640000

_NC = 2
_NS = 16
_NW = _NC * _NS
_EB = 128
_NBLK = -(-N_EDGES // _EB)
_BLK_W = (-(-_NBLK // _NW) + 7) // 8 * 8
_NBLK_PAD = _BLK_W * _NW
_NE_PAD = _NBLK_PAD * _EB
_NPN = 10240
_RPT = _NPN // _NS
_ZR = 128
_BLK_H = _BLK_W // 2


def _scale_rows(rows_ref, val_ref):
    for i in range(_EB):
        sv = val_ref[i]
        for q in range(EMB // 16):
            rows_ref[i, pl.ds(q * 16, 16)] = (
                rows_ref[i, pl.ds(q * 16, 16)] * sv)


def _spmm_body(src_hbm, dst_hbm, val_hbm, emb_hbm, out_hbm,
               acc_spm, src_v, dst_v, val0, val1, rows0, rows1,
               gs0, gs1, vs0, vs1, ss0, ss1):
    cid = lax.axis_index("c")
    sid = lax.axis_index("s")
    wid = cid * _NS + sid
    base = sid * _RPT

    def _zero_row(r, carry):
        for q in range(EMB // 16):
            rows0[r, pl.ds(q * 16, 16)] = jnp.zeros((16,), jnp.float32)
        return carry

    lax.fori_loop(0, _ZR, _zero_row, 0)
    for r in range(_RPT // _ZR):
        pltpu.sync_copy(rows0, acc_spm.at[pl.ds(base + r * _ZR, _ZR)])

    row0 = wid * _BLK_W
    plsc.subcore_barrier()

    _NP = _BLK_H // 2

    def _pair(pi, h0):
        b0 = 2 * pi
        b1 = b0 + 1
        g0 = pltpu.async_copy(emb_hbm.at[src_v.at[b0]], rows0, gs0)
        v0 = pltpu.async_copy(val_hbm.at[h0 + b0], val0, vs0)
        g1 = pltpu.async_copy(emb_hbm.at[src_v.at[b1]], rows1, gs1)
        v1 = pltpu.async_copy(val_hbm.at[h0 + b1], val1, vs1)
        g0.wait()
        v0.wait()
        _scale_rows(rows0, val0)
        s0 = pltpu.async_copy(rows0, acc_spm.at[dst_v.at[b0]], ss0, add=True)
        g1.wait()
        v1.wait()
        _scale_rows(rows1, val1)
        s1 = pltpu.async_copy(rows1, acc_spm.at[dst_v.at[b1]], ss1, add=True)
        s0.wait()
        s1.wait()
        return h0

    for h in range(_BLK_W // _BLK_H):
        h0 = row0 + h * _BLK_H
        pltpu.sync_copy(src_hbm.at[pl.ds(h0, _BLK_H)], src_v)
        pltpu.sync_copy(dst_hbm.at[pl.ds(h0, _BLK_H)], dst_v)
        lax.fori_loop(0, _NP, _pair, h0)

    plsc.subcore_barrier()
    pltpu.sync_copy(acc_spm.at[pl.ds(base, _RPT)],
                    out_hbm.at[cid, pl.ds(base, _RPT)])


@functools.cache
def _get_spmm_sc():
    return pl.kernel(
        _spmm_body,
        out_type=jax.ShapeDtypeStruct((_NC, _NPN, EMB), jnp.float32),
        mesh=plsc.VectorSubcoreMesh(core_axis_name="c", subcore_axis_name="s",
                                    num_cores=_NC, num_subcores=_NS),
        compiler_params=pltpu.CompilerParams(use_tc_tiling_on_sc=False),
        scratch_types=[
            pltpu.VMEM_SHARED((_NPN, EMB), jnp.float32),
            pltpu.VMEM((_BLK_H, _EB), jnp.int32),
            pltpu.VMEM((_BLK_H, _EB), jnp.int32),
            pltpu.VMEM((_EB, 16), jnp.float32),
            pltpu.VMEM((_EB, 16), jnp.float32),
            pltpu.VMEM((_EB, EMB), jnp.float32),
            pltpu.VMEM((_EB, EMB), jnp.float32),
            pltpu.SemaphoreType.DMA,
            pltpu.SemaphoreType.DMA,
            pltpu.SemaphoreType.DMA,
            pltpu.SemaphoreType.DMA,
            pltpu.SemaphoreType.DMA,
            pltpu.SemaphoreType.DMA,
        ],
    )


def _spmm_sc(src, dst, val, e):
    return _get_spmm_sc()(src, dst, val, e)


_BI = 2048
_GI = -(-N_NODE // _BI)
_BJ = 2048
_GJ = -(-N_NODE // _BJ)


def _t1_body(e, avp, avc, p, c, Wi, bi, Wi1, bi1, Wi2, bi2, pt0, pt1, out):
    eb = e[...]
    evp = jnp.dot(avp[...], p[...], preferred_element_type=jnp.float32)
    evc = jnp.dot(avc[...], c[...], preferred_element_type=jnp.float32)
    W = Wi[...]
    z = (jnp.dot(eb, W[0:EMB], preferred_element_type=jnp.float32)
         + jnp.dot(evp, W[EMB:2 * EMB] + Wi1[...],
                   preferred_element_type=jnp.float32)
         + jnp.dot(evc, W[2 * EMB:3 * EMB] + Wi2[...],
                   preferred_element_type=jnp.float32)
         + bi[...] + bi1[...] + bi2[...])
    g = jax.nn.sigmoid(z)
    out[...] = eb + g * evp + (1.0 - g) * evc + pt0[0] + pt1[0]


def _t1(e, p, c, avp, avc, Wi, bi, Wi1, bi1, Wi2, bi2, parts):
    full2 = lambda shape: pl.BlockSpec(shape, lambda i: (0, 0))
    return pl.pallas_call(
        _t1_body,
        grid=(_GI,),
        in_specs=[
            pl.BlockSpec((_BI, EMB), lambda i: (i, 0)),
            pl.BlockSpec((_BI, N_PRICE), lambda i: (i, 0)),
            pl.BlockSpec((_BI, N_CAT), lambda i: (i, 0)),
            full2((N_PRICE, EMB)),
            full2((N_CAT, EMB)),
            full2((3 * EMB, EMB)),
            full2((1, EMB)),
            full2((EMB, EMB)),
            full2((1, EMB)),
            full2((EMB, EMB)),
            full2((1, EMB)),
            pl.BlockSpec((1, _BI, EMB), lambda i: (0, i, 0)),
            pl.BlockSpec((1, _BI, EMB), lambda i: (1, i, 0)),
        ],
        out_specs=pl.BlockSpec((_BI, EMB), lambda i: (i, 0)),
        out_shape=jax.ShapeDtypeStruct((N_NODE, EMB), jnp.float32),
    )(e, avp, avc, p, c, Wi, bi.reshape(1, EMB), Wi1, bi1.reshape(1, EMB),
      Wi2, bi2.reshape(1, EMB), parts, parts)


def _t0_body(e, s_out, smax_out, smin_out, mx, mn):
    j = pl.program_id(0)
    row_ids = jax.lax.broadcasted_iota(jnp.int32, (_BJ, 1), 0) + j * _BJ
    eb = jnp.where(row_ids < N_NODE, e[...], 0.0)
    ones = jnp.ones((1, EMB), jnp.float32)
    s1 = lax.dot_general(ones, eb, (((1,), (1,)), ((), ())),
                         preferred_element_type=jnp.float32)
    s_out[...] = s1
    col_ok = (jax.lax.broadcasted_iota(jnp.int32, (1, _BJ), 1) + j * _BJ) < N_NODE
    bmax = jnp.max(jnp.where(col_ok, s1, -jnp.inf))
    bmin = jnp.min(jnp.where(col_ok, s1, jnp.inf))

    @pl.when(j == 0)
    def _():
        mx[0] = bmax
        mn[0] = bmin

    @pl.when(j > 0)
    def _():
        mx[0] = jnp.maximum(mx[0], bmax)
        mn[0] = jnp.minimum(mn[0], bmin)

    @pl.when(j == _GJ - 1)
    def _():
        smax_out[0, 0] = mx[0]
        smin_out[0, 0] = mn[0]


def _t0(e):
    return pl.pallas_call(
        _t0_body,
        grid=(_GJ,),
        in_specs=[pl.BlockSpec((_BJ, EMB), lambda j: (j, 0))],
        out_specs=[
            pl.BlockSpec((1, _BJ), lambda j: (0, j)),
            pl.BlockSpec(memory_space=pltpu.SMEM),
            pl.BlockSpec(memory_space=pltpu.SMEM),
        ],
        out_shape=[
            jax.ShapeDtypeStruct((1, N_NODE), jnp.float32),
            jax.ShapeDtypeStruct((1, 1), jnp.float32),
            jax.ShapeDtypeStruct((1, 1), jnp.float32),
        ],
        scratch_shapes=[
            pltpu.SMEM((1,), jnp.float32),
            pltpu.SMEM((1,), jnp.float32),
        ],
    )(e)


def _intra_small(A, mat, s_row, e2):
    m = jnp.where(mat > 0, mat * jnp.max(s_row), mat * jnp.min(s_row))
    w = jnp.exp(mat * s_row - m)
    b = w * A
    num = jnp.dot(b, e2, preferred_element_type=jnp.float32)
    den = (jnp.sum(b, axis=1, keepdims=True)
           + 1e-8 * jnp.sum(w, axis=1, keepdims=True))
    return num / den


def _gate_combine(e1, e2, e3, W, b0, W1, b1, W2, b2):
    z = (jnp.dot(e1, W[0:EMB], preferred_element_type=jnp.float32)
         + jnp.dot(e2, W[EMB:2 * EMB] + W1, preferred_element_type=jnp.float32)
         + jnp.dot(e3, W[2 * EMB:3 * EMB] + W2,
                   preferred_element_type=jnp.float32)
         + b0 + b1 + b2)
    g = jax.nn.sigmoid(z)
    return e1 + g * e2 + (1.0 - g) * e3


def _t2_body(apv, acv, e, s, smax, smin, apc, acp, p, c, mpv, mpc, mcp, mcv,
             Wp, bp, Wp1, bp1, Wp2, bp2, Wc, bc, Wc1, bc1, Wc2, bc2,
             price, cate, npv, dpv, spv, ncv, dcv, scv):
    j = pl.program_id(0)

    @pl.when(j == 0)
    def _():
        npv[...] = jnp.zeros_like(npv)
        dpv[...] = jnp.zeros_like(dpv)
        spv[...] = jnp.zeros_like(spv)
        ncv[...] = jnp.zeros_like(ncv)
        dcv[...] = jnp.zeros_like(dcv)
        scv[...] = jnp.zeros_like(scv)

    row_ids = jax.lax.broadcasted_iota(jnp.int32, (_BJ, 1), 0) + j * _BJ
    row_ok = row_ids < N_NODE
    eb = jnp.where(row_ok, e[...], 0.0)
    col_ok = (jax.lax.broadcasted_iota(jnp.int32, (1, _BJ), 1) + j * _BJ) < N_NODE
    s1 = s[...]
    sx = smax[0, 0]
    sn = smin[0, 0]

    m_pv = jnp.where(mpv[...] > 0, mpv[...] * sx, mpv[...] * sn)
    w_pv = jnp.where(col_ok, jnp.exp(mpv[...] * s1 - m_pv), 0.0)
    b_pv = jnp.where(col_ok, w_pv * apv[...], 0.0)
    npv[...] += jnp.dot(b_pv, eb, preferred_element_type=jnp.float32)
    dpv[...] += jnp.sum(b_pv, axis=1, keepdims=True)
    spv[...] += jnp.sum(w_pv, axis=1, keepdims=True)

    m_cv = jnp.where(mcv[...] > 0, mcv[...] * sx, mcv[...] * sn)
    w_cv = jnp.where(col_ok, jnp.exp(mcv[...] * s1 - m_cv), 0.0)
    b_cv = jnp.where(col_ok, w_cv * acv[...], 0.0)
    ncv[...] += jnp.dot(b_cv, eb, preferred_element_type=jnp.float32)
    dcv[...] += jnp.sum(b_cv, axis=1, keepdims=True)
    scv[...] += jnp.sum(w_cv, axis=1, keepdims=True)

    @pl.when(j == _GJ - 1)
    def _():
        ip_v = npv[...] / (dpv[...] + 1e-8 * spv[...])
        ic_v = ncv[...] / (dcv[...] + 1e-8 * scv[...])
        pb = p[...]
        cb = c[...]
        onesE = jnp.ones((1, EMB), jnp.float32)
        s_c = lax.dot_general(onesE, cb, (((1,), (1,)), ((), ())),
                              preferred_element_type=jnp.float32)
        s_p = lax.dot_general(onesE, pb, (((1,), (1,)), ((), ())),
                              preferred_element_type=jnp.float32)
        ip_c = _intra_small(apc[...], mpc[...], s_c, cb)
        ic_p = _intra_small(acp[...], mcp[...], s_p, pb)
        price[...] = _gate_combine(pb, ip_v, ip_c, Wp[...], bp[...],
                                   Wp1[...], bp1[...], Wp2[...], bp2[...])
        cate[...] = _gate_combine(cb, ic_p, ic_v, Wc[...], bc[...],
                                  Wc1[...], bc1[...], Wc2[...], bc2[...])


def _t2(e, s, smax, smin, p, c, apv, apc, acp, acv, mpv, mpc, mcp, mcv,
        Wp, bp, Wp1, bp1, Wp2, bp2, Wc, bc, Wc1, bc1, Wc2, bc2):
    full2 = lambda shape: pl.BlockSpec(shape, lambda j: (0, 0))
    return pl.pallas_call(
        _t2_body,
        grid=(_GJ,),
        in_specs=[
            pl.BlockSpec((N_PRICE, _BJ), lambda j: (0, j)),
            pl.BlockSpec((N_CAT, _BJ), lambda j: (0, j)),
            pl.BlockSpec((_BJ, EMB), lambda j: (j, 0)),
            pl.BlockSpec((1, _BJ), lambda j: (0, j)),
            pl.BlockSpec(memory_space=pltpu.SMEM),
            pl.BlockSpec(memory_space=pltpu.SMEM),
            full2((N_PRICE, N_CAT)),
            full2((N_CAT, N_PRICE)),
            full2((N_PRICE, EMB)),
            full2((N_CAT, EMB)),
            full2((N_PRICE, 1)),
            full2((N_PRICE, 1)),
            full2((N_CAT, 1)),
            full2((N_CAT, 1)),
            full2((3 * EMB, EMB)),
            full2((1, EMB)),
            full2((EMB, EMB)),
            full2((1, EMB)),
            full2((EMB, EMB)),
            full2((1, EMB)),
            full2((3 * EMB, EMB)),
            full2((1, EMB)),
            full2((EMB, EMB)),
            full2((1, EMB)),
            full2((EMB, EMB)),
            full2((1, EMB)),
        ],
        out_specs=[
            pl.BlockSpec((N_PRICE, EMB), lambda j: (0, 0)),
            pl.BlockSpec((N_CAT, EMB), lambda j: (0, 0)),
        ],
        out_shape=[
            jax.ShapeDtypeStruct((N_PRICE, EMB), jnp.float32),
            jax.ShapeDtypeStruct((N_CAT, EMB), jnp.float32),
        ],
        scratch_shapes=[
            pltpu.VMEM((N_PRICE, EMB), jnp.float32),
            pltpu.VMEM((N_PRICE, 1), jnp.float32),
            pltpu.VMEM((N_PRICE, 1), jnp.float32),
            pltpu.VMEM((N_CAT, EMB), jnp.float32),
            pltpu.VMEM((N_CAT, 1), jnp.float32),
            pltpu.VMEM((N_CAT, 1), jnp.float32),
        ],
    )(apv, acv, e, s, smax, smin, apc, acp, p, c, mpv, mpc, mcp, mcv,
      Wp, bp.reshape(1, EMB), Wp1, bp1.reshape(1, EMB), Wp2,
      bp2.reshape(1, EMB), Wc, bc.reshape(1, EMB), Wc1, bc1.reshape(1, EMB),
      Wc2, bc2.reshape(1, EMB))


def kernel(adj_indices, adj_values, A_pv, A_vp, A_pc, A_cp, A_cv, A_vc,
           embedding, pri_emb, cate_emb, mat_pv, mat_pc, mat_cp, mat_cv,
           Wi, bi, Wi1, bi1, Wi2, bi2, Wp, bp, Wp1, bp1, Wp2, bp2,
           Wc, bc, Wc1, bc1, Wc2, bc2):
    npad = _NE_PAD - N_EDGES
    pad_idx = jnp.arange(npad, dtype=jnp.int32) % N_NODE
    src = jnp.concatenate([adj_indices[1], pad_idx]).reshape(_NBLK_PAD, _EB)
    dst = jnp.concatenate([adj_indices[0], pad_idx]).reshape(_NBLK_PAD, _EB)
    val = jnp.broadcast_to(
        jnp.concatenate([adj_values, jnp.zeros((npad,), jnp.float32)])
        .reshape(_NBLK_PAD, _EB)[:, :, None], (_NBLK_PAD, _EB, 16))

    e, p, c = embedding, pri_emb, cate_emb
    item = price = None
    for _ in range(2):
        e_pad = jnp.concatenate(
            [e, jnp.zeros((_NPN - N_NODE, EMB), jnp.float32)])
        parts = _spmm_sc(src, dst, val, e_pad)
        s, smax, smin = _t0(e)
        price, cate = _t2(e, s, smax, smin, p, c, A_pv, A_pc, A_cp, A_cv,
                          mat_pv, mat_pc, mat_cp, mat_cv,
                          Wp, bp, Wp1, bp1, Wp2, bp2,
                          Wc, bc, Wc1, bc1, Wc2, bc2)
        item = _t1(e, p, c, A_vp, A_vc, Wi, bi, Wi1, bi1, Wi2, bi2, parts)
        e, p, c = item, price, cate
    return (item, price)


# speedup vs baseline: 10.0147x; 1.0962x over previous
import functools

import jax
import jax.numpy as jnp
from jax import lax
from jax.experimental import pallas as pl
from jax.experimental.pallas import tpu as pltpu
from jax.experimental.pallas import tpu_sc as plsc

EMB = 64
N_NODE = 10000
N_PRICE = 100
N_CAT = 500
N_EDGES = 640000

_NC = 2
_NS = 16
_NW = _NC * _NS
_EB = 128
_NBLK = -(-N_EDGES // _EB)
_BLK_W = (-(-_NBLK // _NW) + 7) // 8 * 8
_NBLK_PAD = _BLK_W * _NW
_NE_PAD = _NBLK_PAD * _EB
_NPN = 10240
_RPT = _NPN // _NS
_ZR = 128
_BLK_H = _BLK_W // 2


def _scale_rows(rows_ref, val_ref):
    for i in range(_EB):
        sv = val_ref[i]
        for q in range(EMB // 16):
            rows_ref[i, pl.ds(q * 16, 16)] = (
                rows_ref[i, pl.ds(q * 16, 16)] * sv)


def _spmm_body(src_hbm, dst_hbm, val_hbm, emb_hbm, out_hbm,
               acc_spm, src_v, dst_v, val0, val1, rows0, rows1,
               gs0, gs1, vs0, vs1, ss0, ss1):
    cid = lax.axis_index("c")
    sid = lax.axis_index("s")
    wid = cid * _NS + sid
    base = sid * _RPT

    def _zero_row(r, carry):
        for q in range(EMB // 16):
            rows0[r, pl.ds(q * 16, 16)] = jnp.zeros((16,), jnp.float32)
        return carry

    lax.fori_loop(0, _ZR, _zero_row, 0)
    for r in range(_RPT // _ZR):
        pltpu.sync_copy(rows0, acc_spm.at[pl.ds(base + r * _ZR, _ZR)])

    row0 = wid * _BLK_W
    plsc.subcore_barrier()

    _NP = _BLK_H // 2

    def _pair(pi, h0):
        b0 = 2 * pi
        b1 = b0 + 1

        @pl.when(pi > 0)
        def _():
            pltpu.make_async_copy(rows1, acc_spm.at[dst_v.at[b1]], ss1).wait()

        pltpu.async_copy(emb_hbm.at[src_v.at[b1]], rows1, gs1)
        pltpu.async_copy(val_hbm.at[h0 + b1], val1, vs1)

        pltpu.make_async_copy(emb_hbm.at[src_v.at[b0]], rows0, gs0).wait()
        pltpu.make_async_copy(val_hbm.at[h0 + b0], val0, vs0).wait()
        _scale_rows(rows0, val0)
        pltpu.async_copy(rows0, acc_spm.at[dst_v.at[b0]], ss0, add=True)

        pltpu.make_async_copy(emb_hbm.at[src_v.at[b1]], rows1, gs1).wait()
        pltpu.make_async_copy(val_hbm.at[h0 + b1], val1, vs1).wait()
        _scale_rows(rows1, val1)
        pltpu.async_copy(rows1, acc_spm.at[dst_v.at[b1]], ss1, add=True)

        pltpu.make_async_copy(rows0, acc_spm.at[dst_v.at[b0]], ss0).wait()

        @pl.when(pi < _NP - 1)
        def _():
            pltpu.async_copy(emb_hbm.at[src_v.at[b0 + 2]], rows0, gs0)
            pltpu.async_copy(val_hbm.at[h0 + b0 + 2], val0, vs0)

        return h0

    for h in range(_BLK_W // _BLK_H):
        h0 = row0 + h * _BLK_H
        pltpu.sync_copy(src_hbm.at[pl.ds(h0, _BLK_H)], src_v)
        pltpu.sync_copy(dst_hbm.at[pl.ds(h0, _BLK_H)], dst_v)
        pltpu.async_copy(emb_hbm.at[src_v.at[0]], rows0, gs0)
        pltpu.async_copy(val_hbm.at[h0], val0, vs0)
        lax.fori_loop(0, _NP, _pair, h0)
        pltpu.make_async_copy(rows1, acc_spm.at[dst_v.at[_BLK_H - 1]],
                              ss1).wait()

    plsc.subcore_barrier()
    pltpu.sync_copy(acc_spm.at[pl.ds(base, _RPT)],
                    out_hbm.at[cid, pl.ds(base, _RPT)])


@functools.cache
def _get_spmm_sc():
    return pl.kernel(
        _spmm_body,
        out_type=jax.ShapeDtypeStruct((_NC, _NPN, EMB), jnp.float32),
        mesh=plsc.VectorSubcoreMesh(core_axis_name="c", subcore_axis_name="s",
                                    num_cores=_NC, num_subcores=_NS),
        compiler_params=pltpu.CompilerParams(use_tc_tiling_on_sc=False),
        scratch_types=[
            pltpu.VMEM_SHARED((_NPN, EMB), jnp.float32),
            pltpu.VMEM((_BLK_H, _EB), jnp.int32),
            pltpu.VMEM((_BLK_H, _EB), jnp.int32),
            pltpu.VMEM((_EB, 16), jnp.float32),
            pltpu.VMEM((_EB, 16), jnp.float32),
            pltpu.VMEM((_EB, EMB), jnp.float32),
            pltpu.VMEM((_EB, EMB), jnp.float32),
            pltpu.SemaphoreType.DMA,
            pltpu.SemaphoreType.DMA,
            pltpu.SemaphoreType.DMA,
            pltpu.SemaphoreType.DMA,
            pltpu.SemaphoreType.DMA,
            pltpu.SemaphoreType.DMA,
        ],
    )


def _spmm_sc(src, dst, val, e):
    return _get_spmm_sc()(src, dst, val, e)


_BI = 2048
_GI = -(-N_NODE // _BI)
_BJ = 2048
_GJ = -(-N_NODE // _BJ)


def _t1_body(e, avp, avc, p, c, Wi, bi, Wi1, bi1, Wi2, bi2, pt0, pt1, out):
    eb = e[...]
    evp = jnp.dot(avp[...], p[...], preferred_element_type=jnp.float32)
    evc = jnp.dot(avc[...], c[...], preferred_element_type=jnp.float32)
    W = Wi[...]
    z = (jnp.dot(eb, W[0:EMB], preferred_element_type=jnp.float32)
         + jnp.dot(evp, W[EMB:2 * EMB] + Wi1[...],
                   preferred_element_type=jnp.float32)
         + jnp.dot(evc, W[2 * EMB:3 * EMB] + Wi2[...],
                   preferred_element_type=jnp.float32)
         + bi[...] + bi1[...] + bi2[...])
    g = jax.nn.sigmoid(z)
    out[...] = eb + g * evp + (1.0 - g) * evc + pt0[0] + pt1[0]


def _t1(e, p, c, avp, avc, Wi, bi, Wi1, bi1, Wi2, bi2, parts):
    full2 = lambda shape: pl.BlockSpec(shape, lambda i: (0, 0))
    return pl.pallas_call(
        _t1_body,
        grid=(_GI,),
        in_specs=[
            pl.BlockSpec((_BI, EMB), lambda i: (i, 0)),
            pl.BlockSpec((_BI, N_PRICE), lambda i: (i, 0)),
            pl.BlockSpec((_BI, N_CAT), lambda i: (i, 0)),
            full2((N_PRICE, EMB)),
            full2((N_CAT, EMB)),
            full2((3 * EMB, EMB)),
            full2((1, EMB)),
            full2((EMB, EMB)),
            full2((1, EMB)),
            full2((EMB, EMB)),
            full2((1, EMB)),
            pl.BlockSpec((1, _BI, EMB), lambda i: (0, i, 0)),
            pl.BlockSpec((1, _BI, EMB), lambda i: (1, i, 0)),
        ],
        out_specs=pl.BlockSpec((_BI, EMB), lambda i: (i, 0)),
        out_shape=jax.ShapeDtypeStruct((N_NODE, EMB), jnp.float32),
    )(e, avp, avc, p, c, Wi, bi.reshape(1, EMB), Wi1, bi1.reshape(1, EMB),
      Wi2, bi2.reshape(1, EMB), parts, parts)


def _t0_body(e, s_out, smax_out, smin_out, mx, mn):
    j = pl.program_id(0)
    row_ids = jax.lax.broadcasted_iota(jnp.int32, (_BJ, 1), 0) + j * _BJ
    eb = jnp.where(row_ids < N_NODE, e[...], 0.0)
    ones = jnp.ones((1, EMB), jnp.float32)
    s1 = lax.dot_general(ones, eb, (((1,), (1,)), ((), ())),
                         preferred_element_type=jnp.float32)
    s_out[...] = s1
    col_ok = (jax.lax.broadcasted_iota(jnp.int32, (1, _BJ), 1) + j * _BJ) < N_NODE
    bmax = jnp.max(jnp.where(col_ok, s1, -jnp.inf))
    bmin = jnp.min(jnp.where(col_ok, s1, jnp.inf))

    @pl.when(j == 0)
    def _():
        mx[0] = bmax
        mn[0] = bmin

    @pl.when(j > 0)
    def _():
        mx[0] = jnp.maximum(mx[0], bmax)
        mn[0] = jnp.minimum(mn[0], bmin)

    @pl.when(j == _GJ - 1)
    def _():
        smax_out[0, 0] = mx[0]
        smin_out[0, 0] = mn[0]


def _t0(e):
    return pl.pallas_call(
        _t0_body,
        grid=(_GJ,),
        in_specs=[pl.BlockSpec((_BJ, EMB), lambda j: (j, 0))],
        out_specs=[
            pl.BlockSpec((1, _BJ), lambda j: (0, j)),
            pl.BlockSpec(memory_space=pltpu.SMEM),
            pl.BlockSpec(memory_space=pltpu.SMEM),
        ],
        out_shape=[
            jax.ShapeDtypeStruct((1, N_NODE), jnp.float32),
            jax.ShapeDtypeStruct((1, 1), jnp.float32),
            jax.ShapeDtypeStruct((1, 1), jnp.float32),
        ],
        scratch_shapes=[
            pltpu.SMEM((1,), jnp.float32),
            pltpu.SMEM((1,), jnp.float32),
        ],
    )(e)


def _intra_small(A, mat, s_row, e2):
    m = jnp.where(mat > 0, mat * jnp.max(s_row), mat * jnp.min(s_row))
    w = jnp.exp(mat * s_row - m)
    b = w * A
    num = jnp.dot(b, e2, preferred_element_type=jnp.float32)
    den = (jnp.sum(b, axis=1, keepdims=True)
           + 1e-8 * jnp.sum(w, axis=1, keepdims=True))
    return num / den


def _gate_combine(e1, e2, e3, W, b0, W1, b1, W2, b2):
    z = (jnp.dot(e1, W[0:EMB], preferred_element_type=jnp.float32)
         + jnp.dot(e2, W[EMB:2 * EMB] + W1, preferred_element_type=jnp.float32)
         + jnp.dot(e3, W[2 * EMB:3 * EMB] + W2,
                   preferred_element_type=jnp.float32)
         + b0 + b1 + b2)
    g = jax.nn.sigmoid(z)
    return e1 + g * e2 + (1.0 - g) * e3


def _t2_body(apv, acv, e, s, smax, smin, apc, acp, p, c, mpv, mpc, mcp, mcv,
             Wp, bp, Wp1, bp1, Wp2, bp2, Wc, bc, Wc1, bc1, Wc2, bc2,
             price, cate, npv, dpv, spv, ncv, dcv, scv):
    j = pl.program_id(0)

    @pl.when(j == 0)
    def _():
        npv[...] = jnp.zeros_like(npv)
        dpv[...] = jnp.zeros_like(dpv)
        spv[...] = jnp.zeros_like(spv)
        ncv[...] = jnp.zeros_like(ncv)
        dcv[...] = jnp.zeros_like(dcv)
        scv[...] = jnp.zeros_like(scv)

    row_ids = jax.lax.broadcasted_iota(jnp.int32, (_BJ, 1), 0) + j * _BJ
    row_ok = row_ids < N_NODE
    eb = jnp.where(row_ok, e[...], 0.0)
    col_ok = (jax.lax.broadcasted_iota(jnp.int32, (1, _BJ), 1) + j * _BJ) < N_NODE
    s1 = s[...]
    sx = smax[0, 0]
    sn = smin[0, 0]

    m_pv = jnp.where(mpv[...] > 0, mpv[...] * sx, mpv[...] * sn)
    w_pv = jnp.where(col_ok, jnp.exp(mpv[...] * s1 - m_pv), 0.0)
    b_pv = jnp.where(col_ok, w_pv * apv[...], 0.0)
    npv[...] += jnp.dot(b_pv, eb, preferred_element_type=jnp.float32)
    dpv[...] += jnp.sum(b_pv, axis=1, keepdims=True)
    spv[...] += jnp.sum(w_pv, axis=1, keepdims=True)

    m_cv = jnp.where(mcv[...] > 0, mcv[...] * sx, mcv[...] * sn)
    w_cv = jnp.where(col_ok, jnp.exp(mcv[...] * s1 - m_cv), 0.0)
    b_cv = jnp.where(col_ok, w_cv * acv[...], 0.0)
    ncv[...] += jnp.dot(b_cv, eb, preferred_element_type=jnp.float32)
    dcv[...] += jnp.sum(b_cv, axis=1, keepdims=True)
    scv[...] += jnp.sum(w_cv, axis=1, keepdims=True)

    @pl.when(j == _GJ - 1)
    def _():
        ip_v = npv[...] / (dpv[...] + 1e-8 * spv[...])
        ic_v = ncv[...] / (dcv[...] + 1e-8 * scv[...])
        pb = p[...]
        cb = c[...]
        onesE = jnp.ones((1, EMB), jnp.float32)
        s_c = lax.dot_general(onesE, cb, (((1,), (1,)), ((), ())),
                              preferred_element_type=jnp.float32)
        s_p = lax.dot_general(onesE, pb, (((1,), (1,)), ((), ())),
                              preferred_element_type=jnp.float32)
        ip_c = _intra_small(apc[...], mpc[...], s_c, cb)
        ic_p = _intra_small(acp[...], mcp[...], s_p, pb)
        price[...] = _gate_combine(pb, ip_v, ip_c, Wp[...], bp[...],
                                   Wp1[...], bp1[...], Wp2[...], bp2[...])
        cate[...] = _gate_combine(cb, ic_p, ic_v, Wc[...], bc[...],
                                  Wc1[...], bc1[...], Wc2[...], bc2[...])


def _t2(e, s, smax, smin, p, c, apv, apc, acp, acv, mpv, mpc, mcp, mcv,
        Wp, bp, Wp1, bp1, Wp2, bp2, Wc, bc, Wc1, bc1, Wc2, bc2):
    full2 = lambda shape: pl.BlockSpec(shape, lambda j: (0, 0))
    return pl.pallas_call(
        _t2_body,
        grid=(_GJ,),
        in_specs=[
            pl.BlockSpec((N_PRICE, _BJ), lambda j: (0, j)),
            pl.BlockSpec((N_CAT, _BJ), lambda j: (0, j)),
            pl.BlockSpec((_BJ, EMB), lambda j: (j, 0)),
            pl.BlockSpec((1, _BJ), lambda j: (0, j)),
            pl.BlockSpec(memory_space=pltpu.SMEM),
            pl.BlockSpec(memory_space=pltpu.SMEM),
            full2((N_PRICE, N_CAT)),
            full2((N_CAT, N_PRICE)),
            full2((N_PRICE, EMB)),
            full2((N_CAT, EMB)),
            full2((N_PRICE, 1)),
            full2((N_PRICE, 1)),
            full2((N_CAT, 1)),
            full2((N_CAT, 1)),
            full2((3 * EMB, EMB)),
            full2((1, EMB)),
            full2((EMB, EMB)),
            full2((1, EMB)),
            full2((EMB, EMB)),
            full2((1, EMB)),
            full2((3 * EMB, EMB)),
            full2((1, EMB)),
            full2((EMB, EMB)),
            full2((1, EMB)),
            full2((EMB, EMB)),
            full2((1, EMB)),
        ],
        out_specs=[
            pl.BlockSpec((N_PRICE, EMB), lambda j: (0, 0)),
            pl.BlockSpec((N_CAT, EMB), lambda j: (0, 0)),
        ],
        out_shape=[
            jax.ShapeDtypeStruct((N_PRICE, EMB), jnp.float32),
            jax.ShapeDtypeStruct((N_CAT, EMB), jnp.float32),
        ],
        scratch_shapes=[
            pltpu.VMEM((N_PRICE, EMB), jnp.float32),
            pltpu.VMEM((N_PRICE, 1), jnp.float32),
            pltpu.VMEM((N_PRICE, 1), jnp.float32),
            pltpu.VMEM((N_CAT, EMB), jnp.float32),
            pltpu.VMEM((N_CAT, 1), jnp.float32),
            pltpu.VMEM((N_CAT, 1), jnp.float32),
        ],
    )(apv, acv, e, s, smax, smin, apc, acp, p, c, mpv, mpc, mcp, mcv,
      Wp, bp.reshape(1, EMB), Wp1, bp1.reshape(1, EMB), Wp2,
      bp2.reshape(1, EMB), Wc, bc.reshape(1, EMB), Wc1, bc1.reshape(1, EMB),
      Wc2, bc2.reshape(1, EMB))


def kernel(adj_indices, adj_values, A_pv, A_vp, A_pc, A_cp, A_cv, A_vc,
           embedding, pri_emb, cate_emb, mat_pv, mat_pc, mat_cp, mat_cv,
           Wi, bi, Wi1, bi1, Wi2, bi2, Wp, bp, Wp1, bp1, Wp2, bp2,
           Wc, bc, Wc1, bc1, Wc2, bc2):
    npad = _NE_PAD - N_EDGES
    pad_idx = jnp.arange(npad, dtype=jnp.int32) % N_NODE
    src = jnp.concatenate([adj_indices[1], pad_idx]).reshape(_NBLK_PAD, _EB)
    dst = jnp.concatenate([adj_indices[0], pad_idx]).reshape(_NBLK_PAD, _EB)
    val = jnp.broadcast_to(
        jnp.concatenate([adj_values, jnp.zeros((npad,), jnp.float32)])
        .reshape(_NBLK_PAD, _EB)[:, :, None], (_NBLK_PAD, _EB, 16))

    e, p, c = embedding, pri_emb, cate_emb
    item = price = None
    for _ in range(2):
        e_pad = jnp.concatenate(
            [e, jnp.zeros((_NPN - N_NODE, EMB), jnp.float32)])
        parts = _spmm_sc(src, dst, val, e_pad)
        s, smax, smin = _t0(e)
        price, cate = _t2(e, s, smax, smin, p, c, A_pv, A_pc, A_cp, A_cv,
                          mat_pv, mat_pc, mat_cp, mat_cv,
                          Wp, bp, Wp1, bp1, Wp2, bp2,
                          Wc, bc, Wc1, bc1, Wc2, bc2)
        item = _t1(e, p, c, A_vp, A_vc, Wi, bi, Wi1, bi1, Wi2, bi2, parts)
        e, p, c = item, price, cate
    return (item, price)
